# baseline (speedup 1.0000x reference)
import functools

import jax
import jax.numpy as jnp
from jax import lax
from jax.experimental import pallas as pl
from jax.experimental.pallas import tpu as pltpu
from jax.experimental.pallas import tpu_sc as plsc

_NC = 2
_NS = 16
_NW = _NC * _NS
_LANES = 16
_CHUNK_P = 48
_IDX_SHIFT = 4
_MV_BLOCK = 32768


def _matvec_body(su_ref, x_ref, out_ref):
    su = su_ref[...]
    x = x_ref[...]
    y = lax.dot_general(
        su, x, (((1,), (0,)), ((), ())),
        preferred_element_type=jnp.float32)
    out_ref[...] = y[None]


@functools.lru_cache(maxsize=None)
def _build_matvec(vocab, dim):
    grid = pl.cdiv(vocab, _MV_BLOCK)
    return pl.pallas_call(
        _matvec_body,
        grid=(grid,),
        in_specs=[
            pl.BlockSpec((1, dim), lambda i: (0, 0)),
            pl.BlockSpec((dim, _MV_BLOCK), lambda i: (0, i)),
        ],
        out_specs=pl.BlockSpec((1, 1, _MV_BLOCK), lambda i: (i, 0, 0)),
        out_shape=jax.ShapeDtypeStruct((grid, 1, _MV_BLOCK), jnp.float32),
    )


@functools.lru_cache(maxsize=None)
def _build_sc_partials(n_phrases, l_v, t_rows):
    phr_t = n_phrases // _NW
    nch = phr_t // _CHUNK_P
    rows_ch = _CHUNK_P * l_v
    srow = phr_t * _LANES // 128

    @functools.partial(
        pl.kernel,
        mesh=plsc.VectorSubcoreMesh(core_axis_name="c", subcore_axis_name="s"),
        compiler_params=pltpu.CompilerParams(use_tc_tiling_on_sc=False),
        out_type=jax.ShapeDtypeStruct((_NW, srow, 128), jnp.float32),
        scratch_types=[
            pltpu.VMEM((nch, rows_ch), jnp.int32),
            pltpu.VMEM((nch, rows_ch), jnp.int32),
            pltpu.VMEM((2, rows_ch, _LANES), jnp.float32),
            pltpu.VMEM((srow, 128), jnp.float32),
            pltpu.SemaphoreType.DMA,
            pltpu.SemaphoreType.DMA,
        ],
    )
    def sc_partials(idx_hbm, t_hbm, out_hbm,
                    idx_v, lane_v, rows_v, scores_v, sem0, sem1):
        wid = lax.axis_index("s") * _NC + lax.axis_index("c")
        sems = (sem0, sem1)

        pltpu.sync_copy(idx_hbm.at[wid], idx_v)
        for jj in range(nch):
            for k in range(rows_ch // _LANES):
                raw = idx_v[jj, pl.ds(k * _LANES, _LANES)]
                lane_v[jj, pl.ds(k * _LANES, _LANES)] = raw & (_LANES - 1)
                idx_v[jj, pl.ds(k * _LANES, _LANES)] = raw >> _IDX_SHIFT
        iota = lax.iota(jnp.int32, _LANES)

        pltpu.async_copy(t_hbm.at[idx_v.at[0]], rows_v.at[0], sem0)

        def outer(g, carry):
            for b in range(2):
                j = g * 2 + b

                @pl.when(j + 1 < nch)
                def _():
                    pltpu.async_copy(
                        t_hbm.at[idx_v.at[j + 1]], rows_v.at[1 - b], sems[1 - b])

                pltpu.make_async_copy(
                    t_hbm.at[idx_v.at[j]], rows_v.at[b], sems[b]).wait()

                for p in range(_CHUNK_P):
                    lid16 = lane_v[j, pl.ds((p * l_v // _LANES) * _LANES, _LANES)]
                    acc = None
                    for l in range(l_v):
                        row = p * l_v + l
                        lid = jnp.broadcast_to(lid16[row % _LANES], (_LANES,))
                        val = jnp.where(iota == lid, rows_v[b, row, :], 0.0)
                        acc = val if acc is None else acc + val
                    scores_v[(_CHUNK_P // 8) * j + p // 8,
                             pl.ds((p % 8) * _LANES, _LANES)] = acc
            return carry

        lax.fori_loop(0, nch // 2, outer, None)
        pltpu.sync_copy(scores_v, out_hbm.at[wid])

    return sc_partials


_PHR_PER_ROW = 128 // _LANES


def _tc_loss_body(n_pos, x_ref, out_ref):
    x = x_ref[...]
    lane_grp = lax.broadcasted_iota(jnp.int32, (128, _PHR_PER_ROW), 0) // _LANES
    col = lax.broadcasted_iota(jnp.int32, (128, _PHR_PER_ROW), 1)
    sel = (lane_grp == col).astype(jnp.float32)
    score = jnp.dot(x, sel, preferred_element_type=jnp.float32)
    rows = score.shape[0]
    pid = (lax.broadcasted_iota(jnp.int32, (rows, _PHR_PER_ROW), 0)
           * _PHR_PER_ROW
           + lax.broadcasted_iota(jnp.int32, (rows, _PHR_PER_ROW), 1))
    z = jnp.where(pid < n_pos, -score, score)
    sp = jnp.maximum(z, 0.0) + jnp.log(1.0 + jnp.exp(-jnp.abs(z)))
    out_ref[0, 0] = jnp.sum(sp)


def kernel(pos_u, pos_v, neg_v, u_weight, v_weight):
    n_pos, l_v = pos_v.shape
    n_neg = neg_v.shape[0]
    l_u = pos_u.shape[0]
    vocab, dim = u_weight.shape
    n_phr = n_pos + n_neg

    uT = u_weight.T
    pos_u = pos_u.astype(jnp.int32)
    su_col = None
    for j in range(l_u):
        col = lax.dynamic_slice(uT, (0, pos_u[j]), (dim, 1))
        su_col = col if su_col is None else su_col + col
    su = (su_col * (1.0 / float(l_u * l_v))).reshape(1, dim)

    t128 = _build_matvec(vocab, dim)(su, v_weight.T)
    t_rows = t128.shape[0] * _MV_BLOCK // _LANES
    t16 = t128.reshape(t_rows, _LANES)

    idx = jnp.concatenate(
        [pos_v.reshape(-1), neg_v.reshape(-1)]).astype(jnp.int32)
    shape3 = (_NW, (n_phr // _NW) // _CHUNK_P, _CHUNK_P * l_v)
    raw = idx.reshape(shape3)

    sc_partials = _build_sc_partials(n_phr, l_v, t_rows)
    partials = sc_partials(raw, t16)

    loss = pl.pallas_call(
        functools.partial(_tc_loss_body, n_pos),
        out_shape=jax.ShapeDtypeStruct((1, 1), jnp.float32),
        out_specs=pl.BlockSpec(memory_space=pltpu.SMEM),
    )(partials.reshape(n_phr // _PHR_PER_ROW, 128))
    return loss[0, 0]


# speedup vs baseline: 1.0458x; 1.0458x over previous
import functools

import jax
import jax.numpy as jnp
from jax import lax
from jax.experimental import pallas as pl
from jax.experimental.pallas import tpu as pltpu
from jax.experimental.pallas import tpu_sc as plsc

_NC = 2
_NS = 16
_NW = _NC * _NS
_LANES = 16
_CHUNK_P = 32
_IDX_SHIFT = 4
_MV_BLOCK = 32768


def _matvec_body(su_ref, x_ref, out_ref):
    su = su_ref[...]
    x = x_ref[...]
    y = lax.dot_general(
        su, x, (((1,), (0,)), ((), ())),
        preferred_element_type=jnp.float32)
    out_ref[...] = y[None]


@functools.lru_cache(maxsize=None)
def _build_matvec(vocab, dim):
    grid = pl.cdiv(vocab, _MV_BLOCK)
    return pl.pallas_call(
        _matvec_body,
        grid=(grid,),
        in_specs=[
            pl.BlockSpec((1, dim), lambda i: (0, 0)),
            pl.BlockSpec((dim, _MV_BLOCK), lambda i: (0, i)),
        ],
        out_specs=pl.BlockSpec((1, 1, _MV_BLOCK), lambda i: (i, 0, 0)),
        out_shape=jax.ShapeDtypeStruct((grid, 1, _MV_BLOCK), jnp.float32),
    )


@functools.lru_cache(maxsize=None)
def _build_sc_partials(n_phrases, l_v, t_rows):
    phr_t = n_phrases // _NW
    nch = phr_t // _CHUNK_P
    rows_ch = _CHUNK_P * l_v
    srow = phr_t * _LANES // 128

    @functools.partial(
        pl.kernel,
        mesh=plsc.VectorSubcoreMesh(core_axis_name="c", subcore_axis_name="s"),
        compiler_params=pltpu.CompilerParams(use_tc_tiling_on_sc=False),
        out_type=jax.ShapeDtypeStruct((_NW, srow, 128), jnp.float32),
        scratch_types=[
            pltpu.VMEM((nch, rows_ch), jnp.int32),
            pltpu.VMEM((nch, rows_ch), jnp.int32),
            pltpu.VMEM((2, rows_ch, _LANES), jnp.float32),
            pltpu.VMEM((srow, 128), jnp.float32),
            pltpu.SemaphoreType.DMA,
            pltpu.SemaphoreType.DMA,
        ],
    )
    def sc_partials(idx_hbm, t_hbm, out_hbm,
                    idx_v, lane_v, rows_v, scores_v, sem0, sem1):
        wid = lax.axis_index("s") * _NC + lax.axis_index("c")
        sems = (sem0, sem1)

        pltpu.sync_copy(idx_hbm.at[wid], idx_v)
        for jj in range(nch):
            for k in range(rows_ch // _LANES):
                raw = idx_v[jj, pl.ds(k * _LANES, _LANES)]
                lane_v[jj, pl.ds(k * _LANES, _LANES)] = raw & (_LANES - 1)
                idx_v[jj, pl.ds(k * _LANES, _LANES)] = raw >> _IDX_SHIFT
        iota = lax.iota(jnp.int32, _LANES)

        pltpu.async_copy(t_hbm.at[idx_v.at[0]], rows_v.at[0], sem0)

        def outer(g, carry):
            for b in range(2):
                j = g * 2 + b

                @pl.when(j + 1 < nch)
                def _():
                    pltpu.async_copy(
                        t_hbm.at[idx_v.at[j + 1]], rows_v.at[1 - b], sems[1 - b])

                pltpu.make_async_copy(
                    t_hbm.at[idx_v.at[j]], rows_v.at[b], sems[b]).wait()

                for p in range(_CHUNK_P):
                    lid16 = lane_v[j, pl.ds((p * l_v // _LANES) * _LANES, _LANES)]
                    acc = None
                    for l in range(l_v):
                        row = p * l_v + l
                        lid = jnp.broadcast_to(lid16[row % _LANES], (_LANES,))
                        val = jnp.where(iota == lid, rows_v[b, row, :], 0.0)
                        acc = val if acc is None else acc + val
                    scores_v[(_CHUNK_P // 8) * j + p // 8,
                             pl.ds((p % 8) * _LANES, _LANES)] = acc
            return carry

        lax.fori_loop(0, nch // 2, outer, None)
        pltpu.sync_copy(scores_v, out_hbm.at[wid])

    return sc_partials


_PHR_PER_ROW = 128 // _LANES


def _tc_loss_body(n_pos, x_ref, out_ref):
    x = x_ref[...]
    lane_grp = lax.broadcasted_iota(jnp.int32, (128, _PHR_PER_ROW), 0) // _LANES
    col = lax.broadcasted_iota(jnp.int32, (128, _PHR_PER_ROW), 1)
    sel = (lane_grp == col).astype(jnp.float32)
    score = jnp.dot(x, sel, preferred_element_type=jnp.float32)
    rows = score.shape[0]
    pid = (lax.broadcasted_iota(jnp.int32, (rows, _PHR_PER_ROW), 0)
           * _PHR_PER_ROW
           + lax.broadcasted_iota(jnp.int32, (rows, _PHR_PER_ROW), 1))
    z = jnp.where(pid < n_pos, -score, score)
    sp = jnp.maximum(z, 0.0) + jnp.log(1.0 + jnp.exp(-jnp.abs(z)))
    out_ref[0, 0] = jnp.sum(sp)


def kernel(pos_u, pos_v, neg_v, u_weight, v_weight):
    n_pos, l_v = pos_v.shape
    n_neg = neg_v.shape[0]
    l_u = pos_u.shape[0]
    vocab, dim = u_weight.shape
    n_phr = n_pos + n_neg

    uT = u_weight.T
    pos_u = pos_u.astype(jnp.int32)
    su_col = None
    for j in range(l_u):
        col = lax.dynamic_slice(uT, (0, pos_u[j]), (dim, 1))
        su_col = col if su_col is None else su_col + col
    su = (su_col * (1.0 / float(l_u * l_v))).reshape(1, dim)

    t128 = _build_matvec(vocab, dim)(su, v_weight.T)
    t_rows = t128.shape[0] * _MV_BLOCK // _LANES
    t16 = t128.reshape(t_rows, _LANES)

    idx = jnp.concatenate(
        [pos_v.reshape(-1), neg_v.reshape(-1)]).astype(jnp.int32)
    shape3 = (_NW, (n_phr // _NW) // _CHUNK_P, _CHUNK_P * l_v)
    raw = idx.reshape(shape3)

    sc_partials = _build_sc_partials(n_phr, l_v, t_rows)
    partials = sc_partials(raw, t16)

    loss = pl.pallas_call(
        functools.partial(_tc_loss_body, n_pos),
        out_shape=jax.ShapeDtypeStruct((1, 1), jnp.float32),
        out_specs=pl.BlockSpec(memory_space=pltpu.SMEM),
    )(partials.reshape(n_phr // _PHR_PER_ROW, 128))
    return loss[0, 0]
